# baseline (speedup 1.0000x reference)
import jax
import jax.numpy as jnp
from jax import lax
from jax.experimental import pallas as pl
from jax.experimental.pallas import tpu as pltpu
from jax.experimental.pallas import tpu_sc as plsc

N = 10000
E = 320000
D = 128
H0 = 64
H1 = 32
G = 16

NCORES = 2
NSUB = 16
NW = NCORES * NSUB
NP = 10240
RPT = NP // NSUB
CHUNK = 128
NCH = E // CHUNK
CPW = NCH // NW
XTRA = NCH - CPW * NW

_F32 = jnp.float32
_HIGH = lax.Precision.DEFAULT
_sc_params = pltpu.CompilerParams(needs_layout_passes=False,
                                  use_tc_tiling_on_sc=False)


def _mesh():
  return plsc.VectorSubcoreMesh(core_axis_name="c", subcore_axis_name="s",
                                num_cores=NCORES, num_subcores=NSUB)


def _deg_body(e2d_hbm, out_hbm, hist_v, dbuf_v, tbuf_v):
  cid = lax.axis_index("c")
  sid = lax.axis_index("s")
  wid = cid * NSUB + sid

  @pl.loop(0, NP // 16)
  def _(i):
    hist_v[pl.ds(i * 16, 16)] = jnp.zeros((16,), _F32)

  pltpu.sync_copy(e2d_hbm.at[1, pl.ds(wid * CPW, CPW)], dbuf_v)
  ones = jnp.ones((16,), _F32)

  @pl.loop(0, CPW)
  def _(k):
    @pl.loop(0, CHUNK // 16)
    def _(j):
      plsc.addupdate_scatter(hist_v, [dbuf_v[k, pl.ds(j * 16, 16)]], ones)

  @pl.when(wid < XTRA)
  def _():
    pltpu.sync_copy(e2d_hbm.at[1, NW * CPW + wid], tbuf_v)

    @pl.loop(0, CHUNK // 16)
    def _(j):
      plsc.addupdate_scatter(hist_v, [tbuf_v[pl.ds(j * 16, 16)]], ones)

  pltpu.sync_copy(hist_v, out_hbm.at[wid])


_deg_call = pl.kernel(
    _deg_body,
    out_type=jax.ShapeDtypeStruct((NW, NP), _F32),
    mesh=_mesh(),
    compiler_params=_sc_params,
    scratch_types=[
        pltpu.VMEM((NP,), _F32),
        pltpu.VMEM((CPW, CHUNK), jnp.int32),
        pltpu.VMEM((CHUNK,), jnp.int32),
    ],
)


def _make_conv(width, nbuf):
  NBUF = nbuf

  n_iter = -(-CPW // NBUF) * NBUF

  def body(y_hbm, e2d_hbm, out_hbm, acc_sh, si2_v, di2_v, rows_bufs,
           tis_v, tid_v, gsem_bufs, isem0, isem1):
    cid = lax.axis_index("c")
    sid = lax.axis_index("s")
    wid = cid * NSUB + sid
    r0 = sid * RPT

    rows = list(rows_bufs)
    gsems = list(gsem_bufs)

    c0 = wid * CPW
    pltpu.async_copy(e2d_hbm.at[0, pl.ds(c0, CPW)], si2_v, isem0)
    pltpu.async_copy(e2d_hbm.at[1, pl.ds(c0, CPW)], di2_v, isem1)

    @pl.loop(0, CHUNK)
    def _(r):
      @pl.loop(0, width // 16)
      def _(c):
        rows[0][r, pl.ds(c * 16, 16)] = jnp.zeros((16,), _F32)

    for j in range(RPT // CHUNK):
      pltpu.sync_copy(rows[0], acc_sh.at[pl.ds(r0 + j * CHUNK, CHUNK)])
    pltpu.make_async_copy(e2d_hbm.at[0, pl.ds(c0, CPW)], si2_v, isem0).wait()
    pltpu.make_async_copy(e2d_hbm.at[1, pl.ds(c0, CPW)], di2_v, isem1).wait()
    plsc.subcore_barrier()

    for b in range(NBUF):
      pltpu.async_copy(y_hbm.at[si2_v.at[b]], rows[b], gsems[b])

    @pl.loop(0, n_iter, step=NBUF)
    def _(k0):
      for b in range(NBUF):
        k = k0 + b

        @pl.when(k < CPW)
        def _():
          pltpu.make_async_copy(y_hbm.at[si2_v.at[k]], rows[b], gsems[b]).wait()
          pltpu.sync_copy(rows[b], acc_sh.at[di2_v.at[k]], add=True)

          @pl.when(k + NBUF < CPW)
          def _():
            pltpu.async_copy(y_hbm.at[si2_v.at[k + NBUF]], rows[b], gsems[b])

    @pl.when(wid < XTRA)
    def _():
      ce = NW * CPW + wid
      pltpu.sync_copy(e2d_hbm.at[0, ce], tis_v)
      pltpu.sync_copy(e2d_hbm.at[1, ce], tid_v)
      pltpu.async_copy(y_hbm.at[tis_v], rows[0], gsems[0]).wait()
      pltpu.sync_copy(rows[0], acc_sh.at[tid_v], add=True)

    plsc.subcore_barrier()
    pltpu.sync_copy(acc_sh.at[pl.ds(r0, RPT)], out_hbm.at[cid, pl.ds(r0, RPT)])

  return pl.kernel(
      body,
      out_type=jax.ShapeDtypeStruct((NCORES, NP, width), _F32),
      mesh=_mesh(),
      compiler_params=_sc_params,
      scratch_types=[
          pltpu.VMEM_SHARED((NP, width), _F32),
          pltpu.VMEM((CPW, CHUNK), jnp.int32),
          pltpu.VMEM((CPW, CHUNK), jnp.int32),
          tuple(pltpu.VMEM((CHUNK, width), _F32) for _ in range(NBUF)),
          pltpu.VMEM((CHUNK,), jnp.int32),
          pltpu.VMEM((CHUNK,), jnp.int32),
          tuple(pltpu.SemaphoreType.DMA for _ in range(NBUF)),
          pltpu.SemaphoreType.DMA,
          pltpu.SemaphoreType.DMA,
      ],
  )


_conv64 = _make_conv(H0, 8)
_conv32 = _make_conv(H1, 8)


def _mm_body(a_ref, b_ref, o_ref):
  o_ref[...] = lax.dot_general(
      a_ref[...], b_ref[...], (((1,), (0,)), ((), ())),
      precision=_HIGH, preferred_element_type=_F32)


def _tc_mm(a, b):
  return pl.pallas_call(
      _mm_body,
      out_shape=jax.ShapeDtypeStruct((a.shape[0], b.shape[1]), _F32),
  )(a, b)


def _scale_body(xw_ref, degp_ref, y_ref, dinv_ref):
  deg = jnp.sum(degp_ref[...], axis=0)[:, None] + 1.0
  dinv = lax.rsqrt(deg)
  dinv_ref[...] = dinv
  y_ref[...] = dinv * xw_ref[...]


_tc_scale = pl.pallas_call(
    _scale_body,
    out_shape=(
        jax.ShapeDtypeStruct((N, H0), _F32),
        jax.ShapeDtypeStruct((N, 1), _F32),
    ),
)


def _mid_body(accp_hbm, y0_ref, dinv_ref, b0_ref, g0_ref, be0_ref, w1_ref,
              y1_ref, accp_ref, sem):
  pltpu.make_async_copy(accp_hbm, accp_ref, sem).start()
  pltpu.make_async_copy(accp_hbm, accp_ref, sem).wait()
  acc = accp_ref[0, pl.ds(0, N), :] + accp_ref[1, pl.ds(0, N), :]
  dinv = dinv_ref[...]
  h = dinv * (acc + y0_ref[...]) + b0_ref[...]
  mu = jnp.mean(h, axis=0, keepdims=True)
  var = jnp.mean((h - mu) ** 2, axis=0, keepdims=True)
  h = jnp.maximum((h - mu) * lax.rsqrt(var + 1e-5) * g0_ref[...] + be0_ref[...],
                  0.0)
  xw1 = lax.dot_general(h, w1_ref[...], (((1,), (0,)), ((), ())),
                        precision=_HIGH, preferred_element_type=_F32)
  y1_ref[...] = dinv * xw1


_tc_mid = pl.pallas_call(
    _mid_body,
    in_specs=[pl.BlockSpec(memory_space=pltpu.HBM)] + [pl.BlockSpec()] * 6,
    scratch_shapes=[pltpu.VMEM((NCORES, NP, H0), _F32),
                    pltpu.SemaphoreType.DMA],
    out_shape=jax.ShapeDtypeStruct((N, H1), _F32),
)


def _head_body(accp_ref, y1_ref, dinv_ref, b1_ref, g1_ref, be1_ref,
               batch_ref, wc1_ref, bc1_ref, wc2_ref, bc2_ref, o_ref):
  acc = accp_ref[0, pl.ds(0, N), :] + accp_ref[1, pl.ds(0, N), :]
  h = dinv_ref[...] * (acc + y1_ref[...]) + b1_ref[...]
  mu = jnp.mean(h, axis=0, keepdims=True)
  var = jnp.mean((h - mu) ** 2, axis=0, keepdims=True)
  h = jnp.maximum((h - mu) * lax.rsqrt(var + 1e-5) * g1_ref[...] + be1_ref[...],
                  0.0)
  gi = lax.broadcasted_iota(jnp.int32, (1, G), 1)
  onehot = (batch_ref[...] == gi).astype(_F32)
  sums = lax.dot_general(onehot, h, (((0,), (0,)), ((), ())),
                         precision=_HIGH, preferred_element_type=_F32)
  counts = jnp.sum(onehot, axis=0)[:, None]
  pooled = sums / jnp.maximum(counts, 1.0)
  z = jnp.maximum(
      lax.dot_general(pooled, wc1_ref[...], (((1,), (0,)), ((), ())),
                      precision=_HIGH, preferred_element_type=_F32)
      + bc1_ref[...], 0.0)
  o_ref[...] = lax.dot_general(z, wc2_ref[...], (((1,), (0,)), ((), ())),
                               precision=_HIGH, preferred_element_type=_F32) \
      + bc2_ref[...]


_tc_head = pl.pallas_call(
    _head_body,
    out_shape=jax.ShapeDtypeStruct((G, 2), _F32),
)


def kernel(x, edge_index, batch, W0, b0, g0, be0, W1, b1, g1, be1,
           Wc1, bc1, Wc2, bc2):
  e2d = edge_index.reshape(2, NCH, CHUNK)

  degp = _deg_call(e2d)
  xw0 = _tc_mm(x, W0)
  y0, dinv = _tc_scale(xw0, degp[:, :N])
  accp0 = _conv64(y0, e2d)
  y1 = _tc_mid(accp0, y0, dinv, b0[None, :], g0[None, :], be0[None, :], W1)
  accp1 = _conv32(y1, e2d)
  return _tc_head(accp1, y1, dinv, b1[None, :], g1[None, :], be1[None, :],
                  batch[:, None], Wc1, bc1[None, :], Wc2, bc2[None, :])


# speedup vs baseline: 1.0061x; 1.0061x over previous
import jax
import jax.numpy as jnp
from jax import lax
from jax.experimental import pallas as pl
from jax.experimental.pallas import tpu as pltpu
from jax.experimental.pallas import tpu_sc as plsc

N = 10000
E = 320000
D = 128
H0 = 64
H1 = 32
G = 16

NCORES = 2
NSUB = 16
NW = NCORES * NSUB
NP = 10240
RPT = NP // NSUB
CHUNK = 128
NCH = E // CHUNK
CPW = NCH // NW
XTRA = NCH - CPW * NW

_F32 = jnp.float32
_HIGH = lax.Precision.DEFAULT
_sc_params = pltpu.CompilerParams(needs_layout_passes=False,
                                  use_tc_tiling_on_sc=False)


def _mesh():
  return plsc.VectorSubcoreMesh(core_axis_name="c", subcore_axis_name="s",
                                num_cores=NCORES, num_subcores=NSUB)


def _deg_body(e2d_hbm, out_hbm, hist_v, dbuf_v, tbuf_v):
  cid = lax.axis_index("c")
  sid = lax.axis_index("s")
  wid = cid * NSUB + sid

  @pl.loop(0, NP // 16)
  def _(i):
    hist_v[pl.ds(i * 16, 16)] = jnp.zeros((16,), _F32)

  pltpu.sync_copy(e2d_hbm.at[1, pl.ds(wid * CPW, CPW)], dbuf_v)
  ones = jnp.ones((16,), _F32)

  @pl.loop(0, CPW)
  def _(k):
    @pl.loop(0, CHUNK // 16)
    def _(j):
      plsc.addupdate_scatter(hist_v, [dbuf_v[k, pl.ds(j * 16, 16)]], ones)

  @pl.when(wid < XTRA)
  def _():
    pltpu.sync_copy(e2d_hbm.at[1, NW * CPW + wid], tbuf_v)

    @pl.loop(0, CHUNK // 16)
    def _(j):
      plsc.addupdate_scatter(hist_v, [tbuf_v[pl.ds(j * 16, 16)]], ones)

  pltpu.sync_copy(hist_v, out_hbm.at[wid])


_deg_call = pl.kernel(
    _deg_body,
    out_type=jax.ShapeDtypeStruct((NW, NP), _F32),
    mesh=_mesh(),
    compiler_params=_sc_params,
    scratch_types=[
        pltpu.VMEM((NP,), _F32),
        pltpu.VMEM((CPW, CHUNK), jnp.int32),
        pltpu.VMEM((CHUNK,), jnp.int32),
    ],
)


def _make_conv(width, nbuf):
  NBUF = nbuf

  n_iter = -(-CPW // NBUF) * NBUF

  def body(y_hbm, e2d_hbm, out_hbm, acc_sh, si2_v, di2_v, rows_bufs,
           tis_v, tid_v, gsem_bufs, isem0, isem1):
    cid = lax.axis_index("c")
    sid = lax.axis_index("s")
    wid = cid * NSUB + sid
    r0 = sid * RPT

    rows = list(rows_bufs)
    gsems = list(gsem_bufs)

    c0 = wid * CPW
    pltpu.async_copy(e2d_hbm.at[0, pl.ds(c0, CPW)], si2_v, isem0)
    pltpu.async_copy(e2d_hbm.at[1, pl.ds(c0, CPW)], di2_v, isem1)

    @pl.loop(0, CHUNK)
    def _(r):
      @pl.loop(0, width // 16)
      def _(c):
        rows[0][r, pl.ds(c * 16, 16)] = jnp.zeros((16,), _F32)

    for j in range(RPT // CHUNK):
      pltpu.sync_copy(rows[0], acc_sh.at[pl.ds(r0 + j * CHUNK, CHUNK)])
    pltpu.make_async_copy(e2d_hbm.at[0, pl.ds(c0, CPW)], si2_v, isem0).wait()
    pltpu.make_async_copy(e2d_hbm.at[1, pl.ds(c0, CPW)], di2_v, isem1).wait()
    plsc.subcore_barrier()

    for b in range(NBUF):
      pltpu.async_copy(y_hbm.at[si2_v.at[b]], rows[b], gsems[b])

    @pl.loop(0, n_iter, step=NBUF)
    def _(k0):
      for b in range(NBUF):
        k = k0 + b

        @pl.when(k < CPW)
        def _():
          pltpu.make_async_copy(y_hbm.at[si2_v.at[k]], rows[b], gsems[b]).wait()
          pltpu.sync_copy(rows[b], acc_sh.at[di2_v.at[k]], add=True)

          @pl.when(k + NBUF < CPW)
          def _():
            pltpu.async_copy(y_hbm.at[si2_v.at[k + NBUF]], rows[b], gsems[b])

    @pl.when(wid < XTRA)
    def _():
      ce = NW * CPW + wid
      pltpu.sync_copy(e2d_hbm.at[0, ce], tis_v)
      pltpu.sync_copy(e2d_hbm.at[1, ce], tid_v)
      pltpu.async_copy(y_hbm.at[tis_v], rows[0], gsems[0]).wait()
      pltpu.sync_copy(rows[0], acc_sh.at[tid_v], add=True)

    plsc.subcore_barrier()
    pltpu.sync_copy(acc_sh.at[pl.ds(r0, RPT)], out_hbm.at[cid, pl.ds(r0, RPT)])

  return pl.kernel(
      body,
      out_type=jax.ShapeDtypeStruct((NCORES, NP, width), _F32),
      mesh=_mesh(),
      compiler_params=_sc_params,
      scratch_types=[
          pltpu.VMEM_SHARED((NP, width), _F32),
          pltpu.VMEM((CPW, CHUNK), jnp.int32),
          pltpu.VMEM((CPW, CHUNK), jnp.int32),
          tuple(pltpu.VMEM((CHUNK, width), _F32) for _ in range(NBUF)),
          pltpu.VMEM((CHUNK,), jnp.int32),
          pltpu.VMEM((CHUNK,), jnp.int32),
          tuple(pltpu.SemaphoreType.DMA for _ in range(NBUF)),
          pltpu.SemaphoreType.DMA,
          pltpu.SemaphoreType.DMA,
      ],
  )


_conv64 = _make_conv(H0, 8)
_conv32 = _make_conv(H1, 8)


def _mm_body(a_ref, b_ref, o_ref):
  o_ref[...] = lax.dot_general(
      a_ref[...], b_ref[...], (((1,), (0,)), ((), ())),
      precision=_HIGH, preferred_element_type=_F32)


def _tc_mm(a, b):
  return pl.pallas_call(
      _mm_body,
      out_shape=jax.ShapeDtypeStruct((a.shape[0], b.shape[1]), _F32),
  )(a, b)


def _scale_body(xw_ref, degp_ref, y_ref, dinv_ref):
  deg = jnp.sum(degp_ref[...], axis=0)[:, None] + 1.0
  dinv = lax.rsqrt(deg)
  dinv_ref[...] = dinv
  y_ref[...] = dinv * xw_ref[...]


_tc_scale = pl.pallas_call(
    _scale_body,
    out_shape=(
        jax.ShapeDtypeStruct((N, H0), _F32),
        jax.ShapeDtypeStruct((N, 1), _F32),
    ),
)


def _mid_body(accp_ref, y0_ref, dinv_ref, b0_ref, g0_ref, be0_ref, w1_ref,
              y1_ref):
  acc = accp_ref[0, pl.ds(0, N), :] + accp_ref[1, pl.ds(0, N), :]
  dinv = dinv_ref[...]
  h = dinv * (acc + y0_ref[...]) + b0_ref[...]
  mu = jnp.mean(h, axis=0, keepdims=True)
  var = jnp.mean((h - mu) ** 2, axis=0, keepdims=True)
  h = jnp.maximum((h - mu) * lax.rsqrt(var + 1e-5) * g0_ref[...] + be0_ref[...],
                  0.0)
  xw1 = lax.dot_general(h, w1_ref[...], (((1,), (0,)), ((), ())),
                        precision=_HIGH, preferred_element_type=_F32)
  y1_ref[...] = dinv * xw1


_tc_mid = pl.pallas_call(
    _mid_body,
    out_shape=jax.ShapeDtypeStruct((N, H1), _F32),
)


def _head_body(accp_ref, y1_ref, dinv_ref, b1_ref, g1_ref, be1_ref,
               batch_ref, wc1_ref, bc1_ref, wc2_ref, bc2_ref, o_ref):
  acc = accp_ref[0, pl.ds(0, N), :] + accp_ref[1, pl.ds(0, N), :]
  h = dinv_ref[...] * (acc + y1_ref[...]) + b1_ref[...]
  mu = jnp.mean(h, axis=0, keepdims=True)
  var = jnp.mean((h - mu) ** 2, axis=0, keepdims=True)
  h = jnp.maximum((h - mu) * lax.rsqrt(var + 1e-5) * g1_ref[...] + be1_ref[...],
                  0.0)
  gi = lax.broadcasted_iota(jnp.int32, (1, G), 1)
  onehot = (batch_ref[...] == gi).astype(_F32)
  sums = lax.dot_general(onehot, h, (((0,), (0,)), ((), ())),
                         precision=_HIGH, preferred_element_type=_F32)
  counts = jnp.sum(onehot, axis=0)[:, None]
  pooled = sums / jnp.maximum(counts, 1.0)
  z = jnp.maximum(
      lax.dot_general(pooled, wc1_ref[...], (((1,), (0,)), ((), ())),
                      precision=_HIGH, preferred_element_type=_F32)
      + bc1_ref[...], 0.0)
  o_ref[...] = lax.dot_general(z, wc2_ref[...], (((1,), (0,)), ((), ())),
                               precision=_HIGH, preferred_element_type=_F32) \
      + bc2_ref[...]


_tc_head = pl.pallas_call(
    _head_body,
    out_shape=jax.ShapeDtypeStruct((G, 2), _F32),
)


def kernel(x, edge_index, batch, W0, b0, g0, be0, W1, b1, g1, be1,
           Wc1, bc1, Wc2, bc2):
  e2d = edge_index.reshape(2, NCH, CHUNK)

  degp = _deg_call(e2d)
  xw0 = _tc_mm(x, W0)
  y0, dinv = _tc_scale(xw0, degp[:, :N])
  accp0 = _conv64(y0, e2d)
  y1 = _tc_mid(accp0, y0, dinv, b0[None, :], g0[None, :], be0[None, :], W1)
  accp1 = _conv32(y1, e2d)
  return _tc_head(accp1, y1, dinv, b1[None, :], g1[None, :], be1[None, :],
                  batch[:, None], Wc1, bc1[None, :], Wc2, bc2[None, :])
